# baseline (speedup 1.0000x reference)
import functools

import jax
import jax.numpy as jnp
from jax import lax
from jax.experimental import pallas as pl
from jax.experimental.pallas import tpu as pltpu
from jax.experimental.pallas import tpu_sc as plsc

N = 10000
E = 320000
F = 128
NC = 2
NSUB = 16
K = 64
SLOTS_A = 6
GA = 3
PD = 4
SLOTS_D = 4
ITERS = 324
EPS = ITERS * K
EP = EPS * NSUB
NA = N + 8
RB = 1000

_MESH = plsc.VectorSubcoreMesh(core_axis_name="c", subcore_axis_name="s")


@functools.partial(
    pl.kernel,
    out_type=jax.ShapeDtypeStruct((NC, 2, N, F), jnp.float32),
    mesh=_MESH,
    scratch_types=(
        [pltpu.VMEM_SHARED((NA, F), jnp.float32),
         pltpu.VMEM((K, F), jnp.float32)]
        + [pltpu.VMEM((K,), jnp.int32) for _ in range(SLOTS_D)]
        + [pltpu.SemaphoreType.DMA for _ in range(2 * SLOTS_D)]
    ),
)
def _sc_degrees(idxs, ones_hbm, zeros_hbm, out, accum_sh, ones_v, *scr):
    didx = scr[0:SLOTS_D]
    isem = scr[SLOTS_D:2 * SLOTS_D]
    ssem = scr[2 * SLOTS_D:3 * SLOTS_D]
    c = lax.axis_index("c")
    s = lax.axis_index("s")
    z = N // 10
    pltpu.sync_copy(ones_hbm, ones_v)

    for p in range(2):
        @pl.when(s < 10)
        def _zero(p=p):
            pltpu.sync_copy(zeros_hbm.at[pl.ds(s * z, z)],
                            accum_sh.at[pl.ds(s * z, z)])

        plsc.subcore_barrier()
        ebase = (2 * c + p) * EP + s * EPS

        def istart(m, u):
            pltpu.async_copy(idxs.at[pl.ds(ebase + m * K, K)], didx[u],
                             isem[u])

        def iwait(m, u):
            pltpu.make_async_copy(idxs.at[pl.ds(ebase + m * K, K)], didx[u],
                                  isem[u]).wait()

        def sstart(u):
            pltpu.async_copy(ones_v, accum_sh.at[didx[u]], ssem[u], add=True)

        def swait(u):
            pltpu.make_async_copy(ones_v, accum_sh.at[didx[u]], ssem[u]).wait()

        istart(0, 0)
        istart(1, 1)

        def body(o, carry):
            for u in range(SLOTS_D):
                m = SLOTS_D * o + u
                iwait(m, u)
                if u < 2:
                    @pl.when(o > 0)
                    def _(u=u):
                        swait((u - 2) % SLOTS_D)
                else:
                    swait((u - 2) % SLOTS_D)
                sstart(u)
                if u < 2:
                    istart(m + 2, (u + 2) % SLOTS_D)
                else:
                    @pl.when(o < ITERS // SLOTS_D - 1)
                    def _(m=m, u=u):
                        istart(m + 2, (u + 2) % SLOTS_D)
            return carry

        lax.fori_loop(0, ITERS // SLOTS_D, body, 0)
        swait((ITERS - 2) % SLOTS_D)
        swait((ITERS - 1) % SLOTS_D)
        plsc.subcore_barrier()

        @pl.when(s < 10)
        def _writeout(p=p):
            pltpu.sync_copy(accum_sh.at[pl.ds(s * z, z)],
                            out.at[c, p, pl.ds(s * z, z)])

        plsc.subcore_barrier()


@functools.partial(
    pl.kernel,
    out_type=jax.ShapeDtypeStruct((NC, N, F), jnp.float32),
    mesh=_MESH,
    scratch_types=(
        [pltpu.VMEM_SHARED((NA, F), jnp.float32)]
        + [pltpu.VMEM((K,), jnp.int32) for _ in range(2 * SLOTS_A)]
        + [pltpu.VMEM((K, F), jnp.float32) for _ in range(SLOTS_A)]
        + [pltpu.SemaphoreType.DMA for _ in range(3 * SLOTS_A)]
    ),
)
def _sc_agg(tables, srcs, dsts, zeros_hbm, out, accum_sh, *scr):
    sidx = scr[0:SLOTS_A]
    didx = scr[SLOTS_A:2 * SLOTS_A]
    rows = scr[2 * SLOTS_A:3 * SLOTS_A]
    isem = scr[3 * SLOTS_A:4 * SLOTS_A]
    gsem = scr[4 * SLOTS_A:5 * SLOTS_A]
    ssem = scr[5 * SLOTS_A:6 * SLOTS_A]

    c = lax.axis_index("c")
    s = lax.axis_index("s")
    z = N // 10

    @pl.when(s < 10)
    def _zero():
        pltpu.sync_copy(zeros_hbm.at[pl.ds(s * z, z)],
                        accum_sh.at[pl.ds(s * z, z)])

    plsc.subcore_barrier()
    ebase = c * EP + s * EPS

    def istart(m, u):
        pltpu.async_copy(srcs.at[pl.ds(ebase + m * K, K)], sidx[u], isem[u])
        pltpu.async_copy(dsts.at[pl.ds(ebase + m * K, K)], didx[u], isem[u])

    def iwait(m, u):
        pltpu.make_async_copy(srcs.at[pl.ds(ebase + m * K, K)], sidx[u],
                              isem[u]).wait()
        pltpu.make_async_copy(dsts.at[pl.ds(ebase + m * K, K)], didx[u],
                              isem[u]).wait()

    def gstart(u):
        pltpu.async_copy(tables.at[sidx[u]], rows[u], gsem[u])

    def gwait(u):
        pltpu.make_async_copy(tables.at[sidx[u]], rows[u], gsem[u]).wait()

    def sstart(u):
        pltpu.async_copy(rows[u], accum_sh.at[didx[u]], ssem[u], add=True)

    def swait(u):
        pltpu.make_async_copy(rows[u], accum_sh.at[didx[u]], ssem[u]).wait()

    for u in range(PD):
        istart(u, u)
    for u in range(GA):
        iwait(u, u)
        gstart(u)

    def body(o, carry):
        for u in range(SLOTS_A):
            m = SLOTS_A * o + u
            gwait(u)
            sstart(u)
            if u < 2:
                @pl.when(o > 0)
                def _(u=u):
                    swait((u - 2) % SLOTS_A)
            else:
                swait((u - 2) % SLOTS_A)
            if u + PD < SLOTS_A:
                istart(m + PD, (u + PD) % SLOTS_A)
            else:
                @pl.when(o < ITERS // SLOTS_A - 1)
                def _(m=m, u=u):
                    istart(m + PD, (u + PD) % SLOTS_A)
            if u + GA < SLOTS_A:
                iwait(m + GA, (u + GA) % SLOTS_A)
                gstart((u + GA) % SLOTS_A)
            else:
                @pl.when(o < ITERS // SLOTS_A - 1)
                def _(m=m, u=u):
                    iwait(m + GA, (u + GA) % SLOTS_A)
                    gstart((u + GA) % SLOTS_A)
        return carry

    lax.fori_loop(0, ITERS // SLOTS_A, body, 0)
    swait((ITERS - 2) % SLOTS_A)
    swait((ITERS - 1) % SLOTS_A)
    plsc.subcore_barrier()

    @pl.when(s < 10)
    def _writeout():
        pltpu.sync_copy(accum_sh.at[pl.ds(s * z, z)],
                        out.at[c, pl.ds(s * z, z)])


def _rsqrt_deg(deg_blk):
    return lax.rsqrt(jnp.maximum(deg_blk, 1.0))


def _tc_prep_body(xs_ref, xd_ref, deg_ref, out_ref):
    r = _rsqrt_deg(deg_ref[...])
    out_ref[0] = xs_ref[...] * r[:, 0:1]
    out_ref[1] = xd_ref[...] * r[:, 2:3]


def _tc_mid_body(a0_ref, a1_ref, deg_ref, wsd_ref, bsd_ref, wds_ref, bds_ref,
                 out_ref):
    r = _rsqrt_deg(deg_ref[...])
    h_d1 = jax.nn.relu(
        jnp.dot(a0_ref[...] * r[:, 1:2], wsd_ref[...],
                preferred_element_type=jnp.float32) + bsd_ref[...])
    h_s1 = jax.nn.relu(
        jnp.dot(a1_ref[...] * r[:, 3:4], wds_ref[...],
                preferred_element_type=jnp.float32) + bds_ref[...])
    out_ref[0] = h_s1 * r[:, 0:1]
    out_ref[1] = h_d1 * r[:, 2:3]


def _tc_final_body(a0_ref, a1_ref, deg_ref, wsd_ref, bsd_ref, wds_ref, bds_ref,
                   hs_ref, hd_ref):
    r = _rsqrt_deg(deg_ref[...])
    hd_ref[...] = jax.nn.relu(
        jnp.dot(a0_ref[...] * r[:, 1:2], wsd_ref[...],
                preferred_element_type=jnp.float32) + bsd_ref[...])
    hs_ref[...] = jax.nn.relu(
        jnp.dot(a1_ref[...] * r[:, 3:4], wds_ref[...],
                preferred_element_type=jnp.float32) + bds_ref[...])


_ROW = lambda i: (i, 0)
_FIX = lambda i: (0, 0)


def _tc_prep(x_s, x_d, degT):
    return pl.pallas_call(
        _tc_prep_body,
        grid=(N // RB,),
        in_specs=[
            pl.BlockSpec((RB, F), _ROW),
            pl.BlockSpec((RB, F), _ROW),
            pl.BlockSpec((RB, 4), _ROW),
        ],
        out_specs=pl.BlockSpec((2, RB, F), lambda i: (0, i, 0)),
        out_shape=jax.ShapeDtypeStruct((2, N, F), jnp.float32),
    )(x_s, x_d, degT)


def _tc_mid(a0, a1, degT, wsd, bsd, wds, bds):
    return pl.pallas_call(
        _tc_mid_body,
        grid=(N // RB,),
        in_specs=[
            pl.BlockSpec((RB, F), _ROW),
            pl.BlockSpec((RB, F), _ROW),
            pl.BlockSpec((RB, 4), _ROW),
            pl.BlockSpec((F, F), _FIX),
            pl.BlockSpec((1, F), _FIX),
            pl.BlockSpec((F, F), _FIX),
            pl.BlockSpec((1, F), _FIX),
        ],
        out_specs=pl.BlockSpec((2, RB, F), lambda i: (0, i, 0)),
        out_shape=jax.ShapeDtypeStruct((2, N, F), jnp.float32),
    )(a0, a1, degT, wsd, bsd, wds, bds)


def _tc_final(a0, a1, degT, wsd, bsd, wds, bds):
    return pl.pallas_call(
        _tc_final_body,
        grid=(N // RB,),
        in_specs=[
            pl.BlockSpec((RB, F), _ROW),
            pl.BlockSpec((RB, F), _ROW),
            pl.BlockSpec((RB, 4), _ROW),
            pl.BlockSpec((F, F), _FIX),
            pl.BlockSpec((1, F), _FIX),
            pl.BlockSpec((F, F), _FIX),
            pl.BlockSpec((1, F), _FIX),
        ],
        out_specs=[pl.BlockSpec((RB, F), _ROW), pl.BlockSpec((RB, F), _ROW)],
        out_shape=[
            jax.ShapeDtypeStruct((N, F), jnp.float32),
            jax.ShapeDtypeStruct((N, F), jnp.float32),
        ],
    )(a0, a1, degT, wsd, bsd, wds, bds)


def kernel(x_s, x_d, edge_index_sd, edge_index_ds, W_sd1, b_sd1, W_ds1, b_ds1,
           W_sd2, b_sd2, W_ds2, b_ds2):
    sd_src = edge_index_sd[0].astype(jnp.int32)
    sd_dst = edge_index_sd[1].astype(jnp.int32)
    ds_src = edge_index_ds[0].astype(jnp.int32)
    ds_dst = edge_index_ds[1].astype(jnp.int32)

    pad_z = jnp.zeros((EP - E,), jnp.int32)
    pad_t = jnp.full((EP - E,), N, jnp.int32)
    srcs_g = jnp.concatenate([sd_src, pad_z, ds_src + N, pad_z])
    dsts_g = jnp.concatenate([sd_dst, pad_t, ds_dst, pad_t])
    idxs_deg = jnp.concatenate([sd_src, pad_t, sd_dst, pad_t,
                                ds_src, pad_t, ds_dst, pad_t])

    zeros_f = jnp.zeros((N, F), jnp.float32)
    ones_k = jnp.ones((K, F), jnp.float32)

    hist = _sc_degrees(idxs_deg, ones_k, zeros_f)
    degT = jnp.stack([hist[0, 0, :, 0], hist[0, 1, :, 0],
                      hist[1, 0, :, 0], hist[1, 1, :, 0]], axis=1)

    t1 = _tc_prep(x_s, x_d, degT).reshape(2 * N, F)
    agg1 = _sc_agg(t1, srcs_g, dsts_g, zeros_f)
    t2 = _tc_mid(agg1[0], agg1[1], degT,
                 W_sd1, b_sd1.reshape(1, F), W_ds1, b_ds1.reshape(1, F))
    agg2 = _sc_agg(t2.reshape(2 * N, F), srcs_g, dsts_g, zeros_f)
    h_s2, h_d2 = _tc_final(agg2[0], agg2[1], degT,
                           W_sd2, b_sd2.reshape(1, F), W_ds2, b_ds2.reshape(1, F))
    return (h_s2, h_d2)


# speedup vs baseline: 2.1799x; 2.1799x over previous
import functools

import jax
import jax.numpy as jnp
from jax import lax
from jax.experimental import pallas as pl
from jax.experimental.pallas import tpu as pltpu
from jax.experimental.pallas import tpu_sc as plsc

N = 10000
E = 320000
F = 128
NC = 2
NSUB = 16
K = 72
SLOTS_A = 5
GA = 2
PD = 3
SLOTS_D = 4
ITERS = 280
EPS = ITERS * K
EP = EPS * NSUB
NA = N + 8
RB = 1000

_MESH = plsc.VectorSubcoreMesh(core_axis_name="c", subcore_axis_name="s")


@functools.partial(
    pl.kernel,
    out_type=jax.ShapeDtypeStruct((NC, 2, N, F), jnp.float32),
    mesh=_MESH,
    scratch_types=(
        [pltpu.VMEM_SHARED((NA, F), jnp.float32),
         pltpu.VMEM((K, F), jnp.float32)]
        + [pltpu.VMEM((K,), jnp.int32) for _ in range(SLOTS_D)]
        + [pltpu.SemaphoreType.DMA for _ in range(2 * SLOTS_D)]
    ),
)
def _sc_degrees(idxs, ones_hbm, zeros_hbm, out, accum_sh, ones_v, *scr):
    didx = scr[0:SLOTS_D]
    isem = scr[SLOTS_D:2 * SLOTS_D]
    ssem = scr[2 * SLOTS_D:3 * SLOTS_D]
    c = lax.axis_index("c")
    s = lax.axis_index("s")
    z = N // 10
    pltpu.sync_copy(ones_hbm, ones_v)

    for p in range(2):
        @pl.when(s < 10)
        def _zero(p=p):
            pltpu.sync_copy(zeros_hbm.at[pl.ds(s * z, z)],
                            accum_sh.at[pl.ds(s * z, z)])

        plsc.subcore_barrier()
        ebase = (2 * c + p) * EP + s * EPS

        def istart(m, u):
            pltpu.async_copy(idxs.at[pl.ds(ebase + m * K, K)], didx[u],
                             isem[u])

        def iwait(m, u):
            pltpu.make_async_copy(idxs.at[pl.ds(ebase + m * K, K)], didx[u],
                                  isem[u]).wait()

        def sstart(u):
            pltpu.async_copy(ones_v, accum_sh.at[didx[u]], ssem[u], add=True)

        def swait(u):
            pltpu.make_async_copy(ones_v, accum_sh.at[didx[u]], ssem[u]).wait()

        istart(0, 0)
        istart(1, 1)

        def body(o, carry):
            for u in range(SLOTS_D):
                m = SLOTS_D * o + u
                iwait(m, u)
                if u < 2:
                    @pl.when(o > 0)
                    def _(u=u):
                        swait((u - 2) % SLOTS_D)
                else:
                    swait((u - 2) % SLOTS_D)
                sstart(u)
                if u < 2:
                    istart(m + 2, (u + 2) % SLOTS_D)
                else:
                    @pl.when(o < ITERS // SLOTS_D - 1)
                    def _(m=m, u=u):
                        istart(m + 2, (u + 2) % SLOTS_D)
            return carry

        lax.fori_loop(0, ITERS // SLOTS_D, body, 0)
        swait((ITERS - 2) % SLOTS_D)
        swait((ITERS - 1) % SLOTS_D)
        plsc.subcore_barrier()

        @pl.when(s < 10)
        def _writeout(p=p):
            pltpu.sync_copy(accum_sh.at[pl.ds(s * z, z)],
                            out.at[c, p, pl.ds(s * z, z)])

        plsc.subcore_barrier()


@functools.partial(
    pl.kernel,
    out_type=jax.ShapeDtypeStruct((NC, N, F), jnp.float32),
    mesh=_MESH,
    scratch_types=(
        [pltpu.VMEM_SHARED((NA, F), jnp.float32)]
        + [pltpu.VMEM((K,), jnp.int32) for _ in range(2 * SLOTS_A)]
        + [pltpu.VMEM((K, F), jnp.float32) for _ in range(SLOTS_A)]
        + [pltpu.SemaphoreType.DMA for _ in range(3 * SLOTS_A)]
    ),
)
def _sc_agg(tables, srcs, dsts, zeros_hbm, out, accum_sh, *scr):
    sidx = scr[0:SLOTS_A]
    didx = scr[SLOTS_A:2 * SLOTS_A]
    rows = scr[2 * SLOTS_A:3 * SLOTS_A]
    isem = scr[3 * SLOTS_A:4 * SLOTS_A]
    gsem = scr[4 * SLOTS_A:5 * SLOTS_A]
    ssem = scr[5 * SLOTS_A:6 * SLOTS_A]

    c = lax.axis_index("c")
    s = lax.axis_index("s")
    z = N // 10

    @pl.when(s < 10)
    def _zero():
        pltpu.sync_copy(zeros_hbm.at[pl.ds(s * z, z)],
                        accum_sh.at[pl.ds(s * z, z)])

    plsc.subcore_barrier()
    ebase = c * EP + s * EPS

    def istart(m, u):
        pltpu.async_copy(srcs.at[pl.ds(ebase + m * K, K)], sidx[u], isem[u])
        pltpu.async_copy(dsts.at[pl.ds(ebase + m * K, K)], didx[u], isem[u])

    def iwait(m, u):
        pltpu.make_async_copy(srcs.at[pl.ds(ebase + m * K, K)], sidx[u],
                              isem[u]).wait()
        pltpu.make_async_copy(dsts.at[pl.ds(ebase + m * K, K)], didx[u],
                              isem[u]).wait()

    def gstart(u):
        pltpu.async_copy(tables.at[sidx[u]], rows[u], gsem[u])

    def gwait(u):
        pltpu.make_async_copy(tables.at[sidx[u]], rows[u], gsem[u]).wait()

    def sstart(u):
        pltpu.async_copy(rows[u], accum_sh.at[didx[u]], ssem[u], add=True)

    def swait(u):
        pltpu.make_async_copy(rows[u], accum_sh.at[didx[u]], ssem[u]).wait()

    for u in range(PD):
        istart(u, u)
    for u in range(GA):
        iwait(u, u)
        gstart(u)

    def body(o, carry):
        for u in range(SLOTS_A):
            m = SLOTS_A * o + u
            gwait(u)
            sstart(u)
            if u < 2:
                @pl.when(o > 0)
                def _(u=u):
                    swait((u - 2) % SLOTS_A)
            else:
                swait((u - 2) % SLOTS_A)
            if u + PD < SLOTS_A:
                istart(m + PD, (u + PD) % SLOTS_A)
            else:
                @pl.when(o < ITERS // SLOTS_A - 1)
                def _(m=m, u=u):
                    istart(m + PD, (u + PD) % SLOTS_A)
            if u + GA < SLOTS_A:
                iwait(m + GA, (u + GA) % SLOTS_A)
                gstart((u + GA) % SLOTS_A)
            else:
                @pl.when(o < ITERS // SLOTS_A - 1)
                def _(m=m, u=u):
                    iwait(m + GA, (u + GA) % SLOTS_A)
                    gstart((u + GA) % SLOTS_A)
        return carry

    lax.fori_loop(0, ITERS // SLOTS_A, body, 0)
    swait((ITERS - 2) % SLOTS_A)
    swait((ITERS - 1) % SLOTS_A)
    plsc.subcore_barrier()

    @pl.when(s < 10)
    def _writeout():
        pltpu.sync_copy(accum_sh.at[pl.ds(s * z, z)],
                        out.at[c, pl.ds(s * z, z)])


def _rsqrt_deg(deg_blk):
    return lax.rsqrt(jnp.maximum(deg_blk, 1.0))


def _tc_prep_body(xs_ref, xd_ref, deg_ref, out_ref):
    r = _rsqrt_deg(deg_ref[...])
    out_ref[0] = xs_ref[...] * r[:, 0:1]
    out_ref[1] = xd_ref[...] * r[:, 2:3]


def _tc_mid_body(a0_ref, a1_ref, deg_ref, wsd_ref, bsd_ref, wds_ref, bds_ref,
                 out_ref):
    r = _rsqrt_deg(deg_ref[...])
    h_d1 = jax.nn.relu(
        jnp.dot(a0_ref[...] * r[:, 1:2], wsd_ref[...],
                preferred_element_type=jnp.float32) + bsd_ref[...])
    h_s1 = jax.nn.relu(
        jnp.dot(a1_ref[...] * r[:, 3:4], wds_ref[...],
                preferred_element_type=jnp.float32) + bds_ref[...])
    out_ref[0] = h_s1 * r[:, 0:1]
    out_ref[1] = h_d1 * r[:, 2:3]


def _tc_final_body(a0_ref, a1_ref, deg_ref, wsd_ref, bsd_ref, wds_ref, bds_ref,
                   hs_ref, hd_ref):
    r = _rsqrt_deg(deg_ref[...])
    hd_ref[...] = jax.nn.relu(
        jnp.dot(a0_ref[...] * r[:, 1:2], wsd_ref[...],
                preferred_element_type=jnp.float32) + bsd_ref[...])
    hs_ref[...] = jax.nn.relu(
        jnp.dot(a1_ref[...] * r[:, 3:4], wds_ref[...],
                preferred_element_type=jnp.float32) + bds_ref[...])


_ROW = lambda i: (i, 0)
_FIX = lambda i: (0, 0)


def _tc_prep(x_s, x_d, degT):
    return pl.pallas_call(
        _tc_prep_body,
        grid=(N // RB,),
        in_specs=[
            pl.BlockSpec((RB, F), _ROW),
            pl.BlockSpec((RB, F), _ROW),
            pl.BlockSpec((RB, 4), _ROW),
        ],
        out_specs=pl.BlockSpec((2, RB, F), lambda i: (0, i, 0)),
        out_shape=jax.ShapeDtypeStruct((2, N, F), jnp.float32),
    )(x_s, x_d, degT)


def _tc_mid(a0, a1, degT, wsd, bsd, wds, bds):
    return pl.pallas_call(
        _tc_mid_body,
        grid=(N // RB,),
        in_specs=[
            pl.BlockSpec((RB, F), _ROW),
            pl.BlockSpec((RB, F), _ROW),
            pl.BlockSpec((RB, 4), _ROW),
            pl.BlockSpec((F, F), _FIX),
            pl.BlockSpec((1, F), _FIX),
            pl.BlockSpec((F, F), _FIX),
            pl.BlockSpec((1, F), _FIX),
        ],
        out_specs=pl.BlockSpec((2, RB, F), lambda i: (0, i, 0)),
        out_shape=jax.ShapeDtypeStruct((2, N, F), jnp.float32),
    )(a0, a1, degT, wsd, bsd, wds, bds)


def _tc_final(a0, a1, degT, wsd, bsd, wds, bds):
    return pl.pallas_call(
        _tc_final_body,
        grid=(N // RB,),
        in_specs=[
            pl.BlockSpec((RB, F), _ROW),
            pl.BlockSpec((RB, F), _ROW),
            pl.BlockSpec((RB, 4), _ROW),
            pl.BlockSpec((F, F), _FIX),
            pl.BlockSpec((1, F), _FIX),
            pl.BlockSpec((F, F), _FIX),
            pl.BlockSpec((1, F), _FIX),
        ],
        out_specs=[pl.BlockSpec((RB, F), _ROW), pl.BlockSpec((RB, F), _ROW)],
        out_shape=[
            jax.ShapeDtypeStruct((N, F), jnp.float32),
            jax.ShapeDtypeStruct((N, F), jnp.float32),
        ],
    )(a0, a1, degT, wsd, bsd, wds, bds)


def kernel(x_s, x_d, edge_index_sd, edge_index_ds, W_sd1, b_sd1, W_ds1, b_ds1,
           W_sd2, b_sd2, W_ds2, b_ds2):
    sd_src = edge_index_sd[0].astype(jnp.int32)
    sd_dst = edge_index_sd[1].astype(jnp.int32)
    ds_src = edge_index_ds[0].astype(jnp.int32)
    ds_dst = edge_index_ds[1].astype(jnp.int32)

    pad_z = jnp.zeros((EP - E,), jnp.int32)
    pad_t = jnp.full((EP - E,), N, jnp.int32)
    srcs_g = jnp.concatenate([sd_src, pad_z, ds_src + N, pad_z])
    dsts_g = jnp.concatenate([sd_dst, pad_t, ds_dst, pad_t])
    idxs_deg = jnp.concatenate([sd_src, pad_t, sd_dst, pad_t,
                                ds_src, pad_t, ds_dst, pad_t])

    zeros_f = jnp.zeros((N, F), jnp.float32)
    ones_k = jnp.ones((K, F), jnp.float32)

    hist = _sc_degrees(idxs_deg, ones_k, zeros_f)
    degT = jnp.stack([hist[0, 0, :, 0], hist[0, 1, :, 0],
                      hist[1, 0, :, 0], hist[1, 1, :, 0]], axis=1)

    t1 = _tc_prep(x_s, x_d, degT).reshape(2 * N, F)
    agg1 = _sc_agg(t1, srcs_g, dsts_g, zeros_f)
    t2 = _tc_mid(agg1[0], agg1[1], degT,
                 W_sd1, b_sd1.reshape(1, F), W_ds1, b_ds1.reshape(1, F))
    agg2 = _sc_agg(t2.reshape(2 * N, F), srcs_g, dsts_g, zeros_f)
    h_s2, h_d2 = _tc_final(agg2[0], agg2[1], degT,
                           W_sd2, b_sd2.reshape(1, F), W_ds2, b_ds2.reshape(1, F))
    return (h_s2, h_d2)
